# baseline (speedup 1.0000x reference)
import functools

import jax
import jax.numpy as jnp
from jax import lax
from jax.experimental import pallas as pl
from jax.experimental.pallas import tpu as pltpu
from jax.experimental.pallas import tpu_sc as plsc

N_NODES = 100000
MEM_DIM = 512
MSG_DIM = 1024
B = 16384

NC, NS = 2, 16
NW = NC * NS
B_PER_W = B // NW
B_PER_CS = B // NS
HALF = N_NODES // NC
ROWS_PER_SUB = 3128
ROWS_LAST = HALF - (NS - 1) * ROWS_PER_SUB
GCHUNK = 64
ROWS_PER_SUBW = 3128
ROWS_LASTW = N_NODES - (NW - 1) * ROWS_PER_SUBW
SRCBUF_PAD = 3136

@functools.cache
def _mesh():
    return plsc.VectorSubcoreMesh(
        core_axis_name="c", subcore_axis_name="s",
        num_cores=NC, num_subcores=NS,
    )


def _flat_wid():
    c = lax.axis_index("c")
    s = lax.axis_index("s")
    return c * NS + s, c, s


def _gather_body(mem_hbm, ids_hbm, out_hbm, map_hbm,
                 idx_v, rows_v, ids_full, srcbuf, sem):
    w, _, _ = _flat_wid()
    base = w * B_PER_W
    pltpu.sync_copy(ids_hbm.at[pl.ds(base, B_PER_W)], idx_v)
    pltpu.sync_copy(ids_hbm, ids_full)

    @pl.loop(0, B_PER_W // GCHUNK)
    def _(j):
        off = j * GCHUNK
        pltpu.async_copy(
            mem_hbm.at[idx_v.at[pl.ds(off, GCHUNK)]], rows_v, sem
        ).wait()
        pltpu.sync_copy(rows_v, out_hbm.at[pl.ds(base + off, GCHUNK)])

    r0 = w * ROWS_PER_SUBW
    rl = jnp.where(w < NW - 1, ROWS_PER_SUBW, ROWS_LASTW)

    @pl.loop(0, SRCBUF_PAD // 16)
    def _(k):
        srcbuf[pl.ds(k * 16, 16)] = jnp.full((16,), -1, jnp.int32)

    lane = lax.broadcasted_iota(jnp.int32, (16,), 0)

    @pl.loop(0, B // 16)
    def _(cidx):
        ids16 = ids_full[pl.ds(cidx * 16, 16)]
        m = (ids16 >= r0) & (ids16 < r0 + rl)
        pop0 = plsc.all_reduce_population_count(m)

        @pl.when(lax.reduce_max(pop0, axes=(0,)) > 0)
        def _():
            i16 = lane + cidx * 16
            idx16 = jnp.clip(ids16 - r0, 0, SRCBUF_PAD - 1)

            def body(_):
                back = plsc.load_gather(srcbuf, [idx16])
                m2 = m & (i16 > back)
                plsc.store_scatter(srcbuf, [idx16], i16, mask=m2)
                pop = plsc.all_reduce_population_count(m2)
                return lax.reduce_max(pop, axes=(0,)) > 0

            lax.while_loop(lambda c: c, body, True)

    @pl.when(w < NW - 1)
    def _():
        pltpu.sync_copy(srcbuf.at[pl.ds(0, ROWS_PER_SUBW)],
                        map_hbm.at[pl.ds(r0, ROWS_PER_SUBW)])

    @pl.when(w == NW - 1)
    def _():
        pltpu.sync_copy(srcbuf.at[pl.ds(0, ROWS_LASTW)],
                        map_hbm.at[pl.ds(r0, ROWS_LASTW)])


def _sc_gather(memory, node_ids):
    kern = pl.kernel(
        _gather_body,
        out_type=(jax.ShapeDtypeStruct((B, MEM_DIM), jnp.float32),
                  jax.ShapeDtypeStruct((N_NODES,), jnp.int32)),
        mesh=_mesh(),
        compiler_params=pltpu.CompilerParams(needs_layout_passes=False),
        scratch_types=[
            pltpu.VMEM((B_PER_W,), jnp.int32),
            pltpu.VMEM((GCHUNK, MEM_DIM), jnp.float32),
            pltpu.VMEM((B,), jnp.int32),
            pltpu.VMEM((SRCBUF_PAD,), jnp.int32),
            pltpu.SemaphoreType.DMA,
        ],
    )
    return kern(memory, node_ids)


def _srcb_body(map_hbm, ids_hbm, ts_hbm, src_out, tsw_out,
               map_v, ts_v, ids_sl, src_sl, tsw_sl):
    w, _, _ = _flat_wid()
    base = w * B_PER_W
    pltpu.sync_copy(map_hbm, map_v)
    pltpu.sync_copy(ts_hbm, ts_v)
    pltpu.sync_copy(ids_hbm.at[pl.ds(base, B_PER_W)], ids_sl)

    @pl.loop(0, B_PER_W // 16)
    def _(k):
        ids16 = ids_sl[pl.ds(k * 16, 16)]
        src16 = plsc.load_gather(map_v, [ids16])
        ts16 = plsc.load_gather(ts_v, [src16])
        src_sl[pl.ds(k * 16, 16)] = src16
        tsw_sl[pl.ds(k * 16, 16)] = ts16

    pltpu.sync_copy(src_sl, src_out.at[pl.ds(base, B_PER_W)])
    pltpu.sync_copy(tsw_sl, tsw_out.at[pl.ds(base, B_PER_W)])


def _sc_srcb(srcmap, node_ids, timestamps):
    kern = pl.kernel(
        _srcb_body,
        out_type=(jax.ShapeDtypeStruct((B,), jnp.int32),
                  jax.ShapeDtypeStruct((B,), jnp.float32)),
        mesh=_mesh(),
        compiler_params=pltpu.CompilerParams(needs_layout_passes=False),
        scratch_types=[
            pltpu.VMEM((N_NODES,), jnp.int32),
            pltpu.VMEM((B,), jnp.float32),
            pltpu.VMEM((B_PER_W,), jnp.int32),
            pltpu.VMEM((B_PER_W,), jnp.int32),
            pltpu.VMEM((B_PER_W,), jnp.float32),
        ],
    )
    return kern(srcmap, node_ids, timestamps)


def _gi_body(x_ref, wi_ref, bi_ref, o_ref):
    x = x_ref[...].astype(jnp.bfloat16)
    gi = lax.dot_general(
        x, wi_ref[...], (((1,), (1,)), ((), ())),
        preferred_element_type=jnp.float32,
    ) + bi_ref[...]
    o_ref[...] = gi.astype(jnp.bfloat16)


def _tc_gi(unique_msg, W_ih, b_ih):
    BM = 1024
    return pl.pallas_call(
        _gi_body,
        grid=(B // BM,),
        in_specs=[
            pl.BlockSpec((BM, MSG_DIM), lambda i: (i, 0)),
            pl.BlockSpec((3 * MEM_DIM, MSG_DIM), lambda i: (0, 0)),
            pl.BlockSpec((1, 3 * MEM_DIM), lambda i: (0, 0)),
        ],
        out_specs=pl.BlockSpec((BM, 3 * MEM_DIM), lambda i: (i, 0)),
        out_shape=jax.ShapeDtypeStruct((B, 3 * MEM_DIM), jnp.bfloat16),
    )(unique_msg, W_ih.astype(jnp.bfloat16), b_ih.reshape(1, -1))


def _gh_body(gi_ref, h_ref, wh_ref, bh_ref, o_ref):
    h = h_ref[...]
    gh = lax.dot_general(
        h.astype(jnp.bfloat16), wh_ref[...], (((1,), (1,)), ((), ())),
        preferred_element_type=jnp.float32,
    ) + bh_ref[...]
    gi = gi_ref[...].astype(jnp.float32)
    M = MEM_DIM
    r = jax.nn.sigmoid(gi[:, :M] + gh[:, :M])
    z = jax.nn.sigmoid(gi[:, M:2 * M] + gh[:, M:2 * M])
    n = jnp.tanh(gi[:, 2 * M:] + r * gh[:, 2 * M:])
    o_ref[...] = (1.0 - z) * n + z * h


def _tc_gh(gi, h, W_hh, b_hh):
    BM = 1024
    return pl.pallas_call(
        _gh_body,
        grid=(B // BM,),
        in_specs=[
            pl.BlockSpec((BM, 3 * MEM_DIM), lambda i: (i, 0)),
            pl.BlockSpec((BM, MEM_DIM), lambda i: (i, 0)),
            pl.BlockSpec((3 * MEM_DIM, MEM_DIM), lambda i: (0, 0)),
            pl.BlockSpec((1, 3 * MEM_DIM), lambda i: (0, 0)),
        ],
        out_specs=pl.BlockSpec((BM, MEM_DIM), lambda i: (i, 0)),
        out_shape=jax.ShapeDtypeStruct((B, MEM_DIM), jnp.float32),
    )(gi, h, W_hh.astype(jnp.bfloat16), b_hh.reshape(1, -1))


def _scatter_body(hnew_hbm, ids_hbm, src_hbm, mem_ref,
                  srcA, srcB, destA, destB, rowsA, rowsB, sem_g, sem_s):
    w, _, _ = _flat_wid()
    base = w * B_PER_W

    @pl.loop(0, B_PER_W // GCHUNK, step=2)
    def _(j):
        for slot, (src_v, dest_v, rows_v) in enumerate(
                ((srcA, destA, rowsA), (srcB, destB, rowsB))):
            off = base + (j + slot) * GCHUNK
            pltpu.sync_copy(src_hbm.at[pl.ds(off, GCHUNK)], src_v)
            pltpu.sync_copy(ids_hbm.at[pl.ds(off, GCHUNK)], dest_v)
            pltpu.async_copy(hnew_hbm.at[src_v], rows_v, sem_g).wait()
            pltpu.async_copy(rows_v, mem_ref.at[dest_v], sem_s).wait()


def _sc_scatter(mem_ref, h_new, node_ids, src_b):
    kern = pl.kernel(
        _scatter_body,
        out_type=(),
        mesh=_mesh(),
        scratch_types=[
            pltpu.VMEM((GCHUNK,), jnp.int32),
            pltpu.VMEM((GCHUNK,), jnp.int32),
            pltpu.VMEM((GCHUNK,), jnp.int32),
            pltpu.VMEM((GCHUNK,), jnp.int32),
            pltpu.VMEM((GCHUNK, MEM_DIM), jnp.float32),
            pltpu.VMEM((GCHUNK, MEM_DIM), jnp.float32),
            pltpu.SemaphoreType.DMA,
            pltpu.SemaphoreType.DMA,
        ],
    )
    kern(h_new, node_ids, src_b, mem_ref)


def _lu_body(lu_hbm, ids_hbm, tsw_hbm, out_hbm, ids_v, tsw_v, lu_buf):
    w, c, s = _flat_wid()
    base = c * HALF + s * ROWS_PER_SUB

    pltpu.sync_copy(ids_hbm, ids_v)
    pltpu.sync_copy(tsw_hbm, tsw_v)

    @pl.when(s < NS - 1)
    def _():
        pltpu.sync_copy(lu_hbm.at[pl.ds(base, ROWS_PER_SUB)],
                        lu_buf.at[pl.ds(0, ROWS_PER_SUB)])

    @pl.when(s == NS - 1)
    def _():
        pltpu.sync_copy(lu_hbm.at[pl.ds(base, ROWS_LAST)],
                        lu_buf.at[pl.ds(0, ROWS_LAST)])

    limit = base + jnp.where(s < NS - 1, ROWS_PER_SUB, ROWS_LAST)

    @pl.loop(0, B // 16)
    def _(cidx):
        ids16 = ids_v[pl.ds(cidx * 16, 16)]
        ts16 = tsw_v[pl.ds(cidx * 16, 16)]
        m = (ids16 >= base) & (ids16 < limit)
        plsc.store_scatter(lu_buf, [ids16 - base], ts16, mask=m)

    @pl.when(s < NS - 1)
    def _():
        pltpu.sync_copy(lu_buf.at[pl.ds(0, ROWS_PER_SUB)],
                        out_hbm.at[pl.ds(base, ROWS_PER_SUB)])

    @pl.when(s == NS - 1)
    def _():
        pltpu.sync_copy(lu_buf.at[pl.ds(0, ROWS_LAST)],
                        out_hbm.at[pl.ds(base, ROWS_LAST)])


def _sc_last_update(last_update, node_ids, ts_w):
    kern = pl.kernel(
        _lu_body,
        out_type=jax.ShapeDtypeStruct((N_NODES,), jnp.float32),
        mesh=_mesh(),
        compiler_params=pltpu.CompilerParams(needs_layout_passes=False),
        scratch_types=[
            pltpu.VMEM((B,), jnp.int32),
            pltpu.VMEM((B,), jnp.float32),
            pltpu.VMEM((ROWS_PER_SUB,), jnp.float32),
        ],
    )
    return kern(last_update, node_ids, ts_w)


def kernel(memory, last_update, node_ids, unique_msg, timestamps,
           W_ih, W_hh, b_ih, b_hh):
    gi = _tc_gi(unique_msg, W_ih, b_ih)
    h, srcmap = _sc_gather(memory, node_ids)
    src_b, ts_w = _sc_srcb(srcmap, node_ids, timestamps)
    out_lu = _sc_last_update(last_update, node_ids, ts_w)
    h_new = _tc_gh(gi, h, W_hh, b_hh)

    mem_ref = jax.new_ref(memory)
    _sc_scatter(mem_ref, h_new, node_ids, src_b)
    out_mem = mem_ref[...]
    return (out_mem, out_lu)


# speedup vs baseline: 1.0327x; 1.0327x over previous
import functools

import jax
import jax.numpy as jnp
from jax import lax
from jax.experimental import pallas as pl
from jax.experimental.pallas import tpu as pltpu
from jax.experimental.pallas import tpu_sc as plsc

N_NODES = 100000
MEM_DIM = 512
MSG_DIM = 1024
B = 16384

NC, NS = 2, 16
NW = NC * NS
B_PER_W = B // NW
B_PER_CS = B // NS
HALF = N_NODES // NC
ROWS_PER_SUB = 3128
ROWS_LAST = HALF - (NS - 1) * ROWS_PER_SUB
GCHUNK = 64
ROWS_PER_SUBW = 3128
ROWS_LASTW = N_NODES - (NW - 1) * ROWS_PER_SUBW
SRCBUF_PAD = 3136

@functools.cache
def _mesh():
    return plsc.VectorSubcoreMesh(
        core_axis_name="c", subcore_axis_name="s",
        num_cores=NC, num_subcores=NS,
    )


def _flat_wid():
    c = lax.axis_index("c")
    s = lax.axis_index("s")
    return c * NS + s, c, s


def _gather_body(mem_hbm, ids_hbm, out_hbm, idx_v, rowsA, rowsB,
                 semA, semB):
    w, _, _ = _flat_wid()
    base = w * B_PER_W
    pltpu.sync_copy(ids_hbm.at[pl.ds(base, B_PER_W)], idx_v)

    slots = ((rowsA, semA), (rowsB, semB))
    nch = B_PER_W // GCHUNK
    for slot, (rows_v, sem) in enumerate(slots):
        pltpu.async_copy(
            mem_hbm.at[idx_v.at[pl.ds(slot * GCHUNK, GCHUNK)]], rows_v, sem)

    @pl.loop(0, nch - 2)
    def _(j):
        for slot, (rows_v, sem) in enumerate(slots):
            @pl.when(lax.rem(j, 2) == slot)
            def _():
                pltpu.make_async_copy(
                    mem_hbm.at[idx_v.at[pl.ds(0, GCHUNK)]], rows_v, sem
                ).wait()
                pltpu.sync_copy(
                    rows_v, out_hbm.at[pl.ds(base + j * GCHUNK, GCHUNK)])
                pltpu.async_copy(
                    mem_hbm.at[idx_v.at[pl.ds((j + 2) * GCHUNK, GCHUNK)]],
                    rows_v, sem)

    for slot, (rows_v, sem) in enumerate(slots):
        pltpu.make_async_copy(
            mem_hbm.at[idx_v.at[pl.ds(0, GCHUNK)]], rows_v, sem).wait()
        pltpu.sync_copy(
            rows_v,
            out_hbm.at[pl.ds(base + (nch - 2 + slot) * GCHUNK, GCHUNK)])


def _sc_gather(memory, node_ids):
    kern = pl.kernel(
        _gather_body,
        out_type=jax.ShapeDtypeStruct((B, MEM_DIM), jnp.float32),
        mesh=_mesh(),
        scratch_types=[
            pltpu.VMEM((B_PER_W,), jnp.int32),
            pltpu.VMEM((GCHUNK, MEM_DIM), jnp.float32),
            pltpu.VMEM((GCHUNK, MEM_DIM), jnp.float32),
            pltpu.SemaphoreType.DMA,
            pltpu.SemaphoreType.DMA,
        ],
    )
    return kern(memory, node_ids)


def _winner_body(ids_hbm, map_hbm, ids_full, srcbuf):
    w, _, _ = _flat_wid()
    pltpu.sync_copy(ids_hbm, ids_full)
    r0 = w * ROWS_PER_SUBW
    rl = jnp.where(w < NW - 1, ROWS_PER_SUBW, ROWS_LASTW)

    @pl.loop(0, SRCBUF_PAD // 16)
    def _(k):
        srcbuf[pl.ds(k * 16, 16)] = jnp.full((16,), -1, jnp.int32)

    lane = lax.broadcasted_iota(jnp.int32, (16,), 0)

    @pl.loop(0, B // 16)
    def _(cidx):
        ids16 = ids_full[pl.ds(cidx * 16, 16)]
        m = (ids16 >= r0) & (ids16 < r0 + rl)
        pop0 = plsc.all_reduce_population_count(m)

        @pl.when(lax.reduce_max(pop0, axes=(0,)) > 0)
        def _():
            i16 = lane + cidx * 16
            idx16 = jnp.clip(ids16 - r0, 0, SRCBUF_PAD - 1)

            def body(_):
                back = plsc.load_gather(srcbuf, [idx16])
                m2 = m & (i16 > back)
                plsc.store_scatter(srcbuf, [idx16], i16, mask=m2)
                pop = plsc.all_reduce_population_count(m2)
                return lax.reduce_max(pop, axes=(0,)) > 0

            lax.while_loop(lambda c: c, body, True)

    @pl.when(w < NW - 1)
    def _():
        pltpu.sync_copy(srcbuf.at[pl.ds(0, ROWS_PER_SUBW)],
                        map_hbm.at[pl.ds(r0, ROWS_PER_SUBW)])

    @pl.when(w == NW - 1)
    def _():
        pltpu.sync_copy(srcbuf.at[pl.ds(0, ROWS_LASTW)],
                        map_hbm.at[pl.ds(r0, ROWS_LASTW)])


def _sc_winner(node_ids):
    kern = pl.kernel(
        _winner_body,
        out_type=jax.ShapeDtypeStruct((N_NODES,), jnp.int32),
        mesh=_mesh(),
        compiler_params=pltpu.CompilerParams(needs_layout_passes=False),
        scratch_types=[
            pltpu.VMEM((B,), jnp.int32),
            pltpu.VMEM((SRCBUF_PAD,), jnp.int32),
        ],
    )
    return kern(node_ids)


def _srcb_body(map_hbm, ids_hbm, ts_hbm, src_out, tsw_out,
               map_v, ts_v, ids_sl, src_sl, tsw_sl):
    w, _, _ = _flat_wid()
    base = w * B_PER_W
    pltpu.sync_copy(map_hbm, map_v)
    pltpu.sync_copy(ts_hbm, ts_v)
    pltpu.sync_copy(ids_hbm.at[pl.ds(base, B_PER_W)], ids_sl)

    @pl.loop(0, B_PER_W // 16)
    def _(k):
        ids16 = ids_sl[pl.ds(k * 16, 16)]
        src16 = plsc.load_gather(map_v, [ids16])
        ts16 = plsc.load_gather(ts_v, [src16])
        src_sl[pl.ds(k * 16, 16)] = src16
        tsw_sl[pl.ds(k * 16, 16)] = ts16

    pltpu.sync_copy(src_sl, src_out.at[pl.ds(base, B_PER_W)])
    pltpu.sync_copy(tsw_sl, tsw_out.at[pl.ds(base, B_PER_W)])


def _sc_srcb(srcmap, node_ids, timestamps):
    kern = pl.kernel(
        _srcb_body,
        out_type=(jax.ShapeDtypeStruct((B,), jnp.int32),
                  jax.ShapeDtypeStruct((B,), jnp.float32)),
        mesh=_mesh(),
        compiler_params=pltpu.CompilerParams(needs_layout_passes=False),
        scratch_types=[
            pltpu.VMEM((N_NODES,), jnp.int32),
            pltpu.VMEM((B,), jnp.float32),
            pltpu.VMEM((B_PER_W,), jnp.int32),
            pltpu.VMEM((B_PER_W,), jnp.int32),
            pltpu.VMEM((B_PER_W,), jnp.float32),
        ],
    )
    return kern(srcmap, node_ids, timestamps)


def _gi_body(x_ref, wi_ref, bi_ref, o_ref):
    x = x_ref[...].astype(jnp.bfloat16)
    gi = lax.dot_general(
        x, wi_ref[...], (((1,), (1,)), ((), ())),
        preferred_element_type=jnp.float32,
    ) + bi_ref[...]
    o_ref[...] = gi.astype(jnp.bfloat16)


def _tc_gi(unique_msg, W_ih, b_ih):
    BM = 1024
    return pl.pallas_call(
        _gi_body,
        grid=(B // BM,),
        in_specs=[
            pl.BlockSpec((BM, MSG_DIM), lambda i: (i, 0)),
            pl.BlockSpec((3 * MEM_DIM, MSG_DIM), lambda i: (0, 0)),
            pl.BlockSpec((1, 3 * MEM_DIM), lambda i: (0, 0)),
        ],
        out_specs=pl.BlockSpec((BM, 3 * MEM_DIM), lambda i: (i, 0)),
        out_shape=jax.ShapeDtypeStruct((B, 3 * MEM_DIM), jnp.bfloat16),
    )(unique_msg, W_ih.astype(jnp.bfloat16), b_ih.reshape(1, -1))


def _gh_body(gi_ref, h_ref, wh_ref, bh_ref, o_ref):
    h = h_ref[...]
    gh = lax.dot_general(
        h.astype(jnp.bfloat16), wh_ref[...], (((1,), (1,)), ((), ())),
        preferred_element_type=jnp.float32,
    ) + bh_ref[...]
    gi = gi_ref[...].astype(jnp.float32)
    M = MEM_DIM
    r = jax.nn.sigmoid(gi[:, :M] + gh[:, :M])
    z = jax.nn.sigmoid(gi[:, M:2 * M] + gh[:, M:2 * M])
    n = jnp.tanh(gi[:, 2 * M:] + r * gh[:, 2 * M:])
    o_ref[...] = (1.0 - z) * n + z * h


def _tc_gh(gi, h, W_hh, b_hh):
    BM = 1024
    return pl.pallas_call(
        _gh_body,
        grid=(B // BM,),
        in_specs=[
            pl.BlockSpec((BM, 3 * MEM_DIM), lambda i: (i, 0)),
            pl.BlockSpec((BM, MEM_DIM), lambda i: (i, 0)),
            pl.BlockSpec((3 * MEM_DIM, MEM_DIM), lambda i: (0, 0)),
            pl.BlockSpec((1, 3 * MEM_DIM), lambda i: (0, 0)),
        ],
        out_specs=pl.BlockSpec((BM, MEM_DIM), lambda i: (i, 0)),
        out_shape=jax.ShapeDtypeStruct((B, MEM_DIM), jnp.float32),
    )(gi, h, W_hh.astype(jnp.bfloat16), b_hh.reshape(1, -1))


def _scatter_body(hnew_hbm, ids_hbm, src_hbm, mem_ref,
                  srcA, srcB, destA, destB, rowsA, rowsB, sem_g, sem_s):
    w, _, _ = _flat_wid()
    base = w * B_PER_W

    @pl.loop(0, B_PER_W // GCHUNK, step=2)
    def _(j):
        for slot, (src_v, dest_v, rows_v) in enumerate(
                ((srcA, destA, rowsA), (srcB, destB, rowsB))):
            off = base + (j + slot) * GCHUNK
            pltpu.sync_copy(src_hbm.at[pl.ds(off, GCHUNK)], src_v)
            pltpu.sync_copy(ids_hbm.at[pl.ds(off, GCHUNK)], dest_v)
            pltpu.async_copy(hnew_hbm.at[src_v], rows_v, sem_g).wait()
            pltpu.async_copy(rows_v, mem_ref.at[dest_v], sem_s).wait()


def _sc_scatter(mem_ref, h_new, node_ids, src_b):
    kern = pl.kernel(
        _scatter_body,
        out_type=(),
        mesh=_mesh(),
        scratch_types=[
            pltpu.VMEM((GCHUNK,), jnp.int32),
            pltpu.VMEM((GCHUNK,), jnp.int32),
            pltpu.VMEM((GCHUNK,), jnp.int32),
            pltpu.VMEM((GCHUNK,), jnp.int32),
            pltpu.VMEM((GCHUNK, MEM_DIM), jnp.float32),
            pltpu.VMEM((GCHUNK, MEM_DIM), jnp.float32),
            pltpu.SemaphoreType.DMA,
            pltpu.SemaphoreType.DMA,
        ],
    )
    kern(h_new, node_ids, src_b, mem_ref)


def _lu_body(lu_hbm, ids_hbm, tsw_hbm, out_hbm, ids_v, tsw_v, lu_buf):
    w, c, s = _flat_wid()
    base = c * HALF + s * ROWS_PER_SUB

    pltpu.sync_copy(ids_hbm, ids_v)
    pltpu.sync_copy(tsw_hbm, tsw_v)

    @pl.when(s < NS - 1)
    def _():
        pltpu.sync_copy(lu_hbm.at[pl.ds(base, ROWS_PER_SUB)],
                        lu_buf.at[pl.ds(0, ROWS_PER_SUB)])

    @pl.when(s == NS - 1)
    def _():
        pltpu.sync_copy(lu_hbm.at[pl.ds(base, ROWS_LAST)],
                        lu_buf.at[pl.ds(0, ROWS_LAST)])

    limit = base + jnp.where(s < NS - 1, ROWS_PER_SUB, ROWS_LAST)

    @pl.loop(0, B // 16)
    def _(cidx):
        ids16 = ids_v[pl.ds(cidx * 16, 16)]
        ts16 = tsw_v[pl.ds(cidx * 16, 16)]
        m = (ids16 >= base) & (ids16 < limit)
        plsc.store_scatter(lu_buf, [ids16 - base], ts16, mask=m)

    @pl.when(s < NS - 1)
    def _():
        pltpu.sync_copy(lu_buf.at[pl.ds(0, ROWS_PER_SUB)],
                        out_hbm.at[pl.ds(base, ROWS_PER_SUB)])

    @pl.when(s == NS - 1)
    def _():
        pltpu.sync_copy(lu_buf.at[pl.ds(0, ROWS_LAST)],
                        out_hbm.at[pl.ds(base, ROWS_LAST)])


def _sc_last_update(last_update, node_ids, ts_w):
    kern = pl.kernel(
        _lu_body,
        out_type=jax.ShapeDtypeStruct((N_NODES,), jnp.float32),
        mesh=_mesh(),
        compiler_params=pltpu.CompilerParams(needs_layout_passes=False),
        scratch_types=[
            pltpu.VMEM((B,), jnp.int32),
            pltpu.VMEM((B,), jnp.float32),
            pltpu.VMEM((ROWS_PER_SUB,), jnp.float32),
        ],
    )
    return kern(last_update, node_ids, ts_w)


def kernel(memory, last_update, node_ids, unique_msg, timestamps,
           W_ih, W_hh, b_ih, b_hh):
    gi = _tc_gi(unique_msg, W_ih, b_ih)
    h = _sc_gather(memory, node_ids)
    srcmap = _sc_winner(node_ids)
    src_b, ts_w = _sc_srcb(srcmap, node_ids, timestamps)
    out_lu = _sc_last_update(last_update, node_ids, ts_w)
    h_new = _tc_gh(gi, h, W_hh, b_hh)

    mem_ref = jax.new_ref(memory)
    _sc_scatter(mem_ref, h_new, node_ids, src_b)
    out_mem = mem_ref[...]
    return (out_mem, out_lu)
